# baseline (speedup 1.0000x reference)
import functools

import jax
import jax.numpy as jnp
from jax.experimental import pallas as pl
from jax.experimental.pallas import tpu as pltpu
from jax.experimental.pallas import tpu_sc as plsc

_V = 100000
_EPS = 0.1
_CONF = 1.0 - _EPS
_SMOOTH = _EPS / (_V - 2)
_IGNORE = 0

_BV = 2048
_NV = (_V + _BV - 1) // _BV
_LANES = 128


def _gather_slivers(table, ridx, n):
    info = plsc.get_sparse_core_info()
    nw = info.num_cores * info.num_subcores
    bpw = n // nw
    mesh = plsc.VectorSubcoreMesh(core_axis_name="c", subcore_axis_name="s")

    @functools.partial(
        pl.kernel,
        mesh=mesh,
        out_type=jax.ShapeDtypeStruct((n, _LANES), jnp.float32),
        scratch_types=[
            pltpu.VMEM((bpw,), jnp.int32),
            pltpu.VMEM((bpw, _LANES), jnp.float32),
            pltpu.SemaphoreType.DMA,
        ],
    )
    def k(table_hbm, idx_hbm, out_hbm, idx_v, rows_v, sem):
        wid = jax.lax.axis_index("s") * info.num_cores + jax.lax.axis_index("c")
        base = wid * bpw
        pltpu.sync_copy(idx_hbm.at[pl.ds(base, bpw)], idx_v)
        pltpu.async_copy(table_hbm.at[idx_v], rows_v, sem).wait()
        pltpu.sync_copy(rows_v, out_hbm.at[pl.ds(base, bpw)])

    return k(table, ridx)


def _loss_kernel(x_ref, t_ref, g_ref, lane_ref, out_ref,
                 m_ref, s_ref, sum_ref, p0_ref):
    j = pl.program_id(0)
    x = x_ref[...]

    @pl.when(j == 0)
    def _():
        blk_max = jnp.max(x, axis=1, keepdims=True)
        m_ref[...] = blk_max
        s_ref[...] = jnp.sum(jnp.exp(x - blk_max), axis=1, keepdims=True)
        sum_ref[...] = jnp.sum(x, axis=1, keepdims=True)
        p0_ref[...] = x[:, 0:1]

    @pl.when(jnp.logical_and(j > 0, j < _NV - 1))
    def _():
        blk_max = jnp.max(x, axis=1, keepdims=True)
        m_old = m_ref[...]
        m_new = jnp.maximum(m_old, blk_max)
        s_ref[...] = s_ref[...] * jnp.exp(m_old - m_new) + jnp.sum(
            jnp.exp(x - m_new), axis=1, keepdims=True)
        m_ref[...] = m_new
        sum_ref[...] = sum_ref[...] + jnp.sum(x, axis=1, keepdims=True)

    @pl.when(j == _NV - 1)
    def _():
        r = x.shape[0]
        col_ids = j * _BV + jax.lax.broadcasted_iota(jnp.int32, (r, _BV), 1)
        valid = col_ids < _V
        xm = jnp.where(valid, x, -jnp.inf)
        blk_max = jnp.max(xm, axis=1, keepdims=True)
        m_old = m_ref[...]
        m_new = jnp.maximum(m_old, blk_max)
        s = s_ref[...] * jnp.exp(m_old - m_new) + jnp.sum(
            jnp.exp(xm - m_new), axis=1, keepdims=True)
        rowsum = sum_ref[...] + jnp.sum(jnp.where(valid, x, 0.0), axis=1,
                                        keepdims=True)

        t = t_ref[...]
        g = g_ref[...]
        lane = lane_ref[...]
        lane_ids = jax.lax.broadcasted_iota(jnp.int32, (r, _LANES), 1)
        pt = jnp.sum(jnp.where(lane_ids == lane, g, 0.0), axis=1,
                     keepdims=True)

        lse = m_new + jnp.log(s)
        logp_t = pt - lse
        logp_0 = p0_ref[...] - lse
        s_row = rowsum - jnp.float32(_V) * lse
        contrib = _SMOOTH * (s_row - logp_t - logp_0) + _CONF * logp_t
        rmask = t != _IGNORE
        contrib = jnp.where(rmask, contrib, 0.0)
        n_valid = jnp.sum(rmask.astype(jnp.float32))
        loss = -jnp.sum(contrib) / jnp.maximum(n_valid, 1.0)
        out_ref[...] = loss.reshape(1, 1)


def kernel(pred, target):
    pred2 = pred.reshape(-1, pred.shape[-1])
    n = pred2.shape[0]
    t = target.reshape(n).astype(jnp.int32)

    flat = jnp.arange(n, dtype=jnp.int32) * _V + t
    ridx = flat // _LANES
    lane = (flat % _LANES).reshape(n, 1)

    slivers = _gather_slivers(pred2.reshape(n * _V // _LANES, _LANES), ridx, n)

    out = pl.pallas_call(
        _loss_kernel,
        grid=(_NV,),
        in_specs=[
            pl.BlockSpec((n, _BV), lambda j: (0, j)),
            pl.BlockSpec((n, 1), lambda j: (0, 0)),
            pl.BlockSpec((n, _LANES), lambda j: (0, 0)),
            pl.BlockSpec((n, 1), lambda j: (0, 0)),
        ],
        out_specs=pl.BlockSpec((1, 1), lambda j: (0, 0)),
        out_shape=jax.ShapeDtypeStruct((1, 1), jnp.float32),
        scratch_shapes=[pltpu.VMEM((n, 1), jnp.float32) for _ in range(4)],
    )(pred2, t.reshape(n, 1), slivers, lane)
    return out[0, 0]


# speedup vs baseline: 2.1441x; 2.1441x over previous
import jax
import jax.numpy as jnp
from jax.experimental import pallas as pl
from jax.experimental.pallas import tpu as pltpu

_V = 100000
_EPS = 0.1
_CONF = 1.0 - _EPS
_SMOOTH = _EPS / (_V - 2)
_IGNORE = 0

_BV = 2048
_NV = (_V + _BV - 1) // _BV
_LANES = 128
_CMAX = (_V - 160) // _LANES * _LANES
_TAIL = _CMAX + _LANES
_RPB = 22


def _loss_kernel(t_sm, x_ref, pred_any, t_ref, out_ref,
                 m_ref, s_ref, sum_ref, p0_ref, sliver_ref, sem):
    j = pl.program_id(0)
    n = sliver_ref.shape[0]
    x = x_ref[...]

    def _sliver_copy(r):
        c = jnp.minimum((t_sm[r] // _LANES) * _LANES, _CMAX)
        return pltpu.make_async_copy(
            pred_any.at[r, pl.ds(c, _LANES)], sliver_ref.at[r], sem)

    base = j * _RPB

    def _issue(r, carry):
        _sliver_copy(r).start()
        return carry

    jax.lax.fori_loop(base, jnp.minimum(base + _RPB, n), _issue, 0)

    @pl.when(j == 0)
    def _():
        blk_max = jnp.max(x, axis=1, keepdims=True)
        m_ref[...] = blk_max
        s_ref[...] = jnp.sum(jnp.exp(x - blk_max), axis=1, keepdims=True)
        sum_ref[...] = jnp.sum(x, axis=1, keepdims=True)
        p0_ref[...] = x[:, 0:1]

    @pl.when(jnp.logical_and(j > 0, j < _NV - 1))
    def _():
        blk_max = jnp.max(x, axis=1, keepdims=True)
        m_old = m_ref[...]
        m_new = jnp.maximum(m_old, blk_max)
        s_ref[...] = s_ref[...] * jnp.exp(m_old - m_new) + jnp.sum(
            jnp.exp(x - m_new), axis=1, keepdims=True)
        m_ref[...] = m_new
        sum_ref[...] = sum_ref[...] + jnp.sum(x, axis=1, keepdims=True)

    @pl.when(j == _NV - 1)
    def _():
        r = x.shape[0]
        col_ids = j * _BV + jax.lax.broadcasted_iota(jnp.int32, (r, _BV), 1)
        valid = col_ids < _V
        xm = jnp.where(valid, x, -jnp.inf)
        blk_max = jnp.max(xm, axis=1, keepdims=True)
        m_old = m_ref[...]
        m_new = jnp.maximum(m_old, blk_max)
        s = s_ref[...] * jnp.exp(m_old - m_new) + jnp.sum(
            jnp.exp(xm - m_new), axis=1, keepdims=True)
        rowsum = sum_ref[...] + jnp.sum(jnp.where(valid, x, 0.0), axis=1,
                                        keepdims=True)

        t = t_ref[...]
        pt_blk = jnp.sum(jnp.where(col_ids == t, x, 0.0), axis=1,
                         keepdims=True)

        def _wait(rr, carry):
            _sliver_copy(rr).wait()
            return carry

        jax.lax.fori_loop(0, n, _wait, 0)
        g = sliver_ref[...]
        c_vec = jnp.minimum((t // _LANES) * _LANES, _CMAX)
        lane = t - c_vec
        lane_ids = jax.lax.broadcasted_iota(jnp.int32, (r, _LANES), 1)
        pt_sliver = jnp.sum(jnp.where(lane_ids == lane, g, 0.0), axis=1,
                            keepdims=True)
        pt = jnp.where(t >= _TAIL, pt_blk, pt_sliver)

        lse = m_new + jnp.log(s)
        logp_t = pt - lse
        logp_0 = p0_ref[...] - lse
        s_row = rowsum - jnp.float32(_V) * lse
        contrib = _SMOOTH * (s_row - logp_t - logp_0) + _CONF * logp_t
        rmask = t != _IGNORE
        contrib = jnp.where(rmask, contrib, 0.0)
        n_valid = jnp.sum(rmask.astype(jnp.float32))
        loss = -jnp.sum(contrib) / jnp.maximum(n_valid, 1.0)
        out_ref[...] = loss.reshape(1, 1)


def kernel(pred, target):
    pred2 = pred.reshape(-1, pred.shape[-1])
    n = pred2.shape[0]
    t = target.reshape(n).astype(jnp.int32)

    grid_spec = pltpu.PrefetchScalarGridSpec(
        num_scalar_prefetch=1,
        grid=(_NV,),
        in_specs=[
            pl.BlockSpec((n, _BV), lambda j, t_sm: (0, j)),
            pl.BlockSpec(memory_space=pltpu.MemorySpace.HBM),
            pl.BlockSpec((n, 1), lambda j, t_sm: (0, 0)),
        ],
        out_specs=pl.BlockSpec((1, 1), lambda j, t_sm: (0, 0)),
        scratch_shapes=[
            pltpu.VMEM((n, 1), jnp.float32),
            pltpu.VMEM((n, 1), jnp.float32),
            pltpu.VMEM((n, 1), jnp.float32),
            pltpu.VMEM((n, 1), jnp.float32),
            pltpu.VMEM((n, _LANES), jnp.float32),
            pltpu.SemaphoreType.DMA,
        ],
    )
    out = pl.pallas_call(
        _loss_kernel,
        grid_spec=grid_spec,
        out_shape=jax.ShapeDtypeStruct((1, 1), jnp.float32),
    )(t, pred2, pred2, t.reshape(n, 1))
    return out[0, 0]


# speedup vs baseline: 2.1557x; 1.0054x over previous
import jax
import jax.numpy as jnp
from jax.experimental import pallas as pl
from jax.experimental.pallas import tpu as pltpu

_V = 100000
_EPS = 0.1
_CONF = 1.0 - _EPS
_SMOOTH = _EPS / (_V - 2)
_IGNORE = 0

_LANES = 128
_VA = _V // _LANES * _LANES
_CMAX = (_V - 160) // _LANES * _LANES
_TAIL = _CMAX + _LANES
_BR = 32


def _loss_kernel(t_sm, x_ref, pred_any, t_ref, out_ref,
                 sliver_ref, sem, acc_ref, nv_ref):
    bi = pl.program_id(0)
    nb = pl.num_programs(0)
    base = bi * _BR

    def _sliver_copy(local):
        r = base + local
        c = jnp.minimum((t_sm[r] // _LANES) * _LANES, _CMAX)
        return pltpu.make_async_copy(
            pred_any.at[r, pl.ds(c, _LANES)], sliver_ref.at[local], sem)

    def _issue(local, carry):
        _sliver_copy(local).start()
        return carry

    jax.lax.fori_loop(0, _BR, _issue, 0)

    x = x_ref[...]
    xa = x[:, :_VA]
    xt = x[:, _VA:_V]
    m = jnp.maximum(jnp.max(xa, axis=1, keepdims=True),
                    jnp.max(xt, axis=1, keepdims=True))
    rowsum = (jnp.sum(xa, axis=1, keepdims=True)
              + jnp.sum(xt, axis=1, keepdims=True))
    es = (jnp.sum(jnp.exp(xa - m), axis=1, keepdims=True)
          + jnp.sum(jnp.exp(xt - m), axis=1, keepdims=True))
    p0 = x[:, 0:1]

    t = t_ref[...]
    tail_ids = _VA + jax.lax.broadcasted_iota(jnp.int32, (_BR, _V - _VA), 1)
    pt_tail = jnp.sum(jnp.where(tail_ids == t, xt, 0.0), axis=1,
                      keepdims=True)

    def _wait(local, carry):
        _sliver_copy(local).wait()
        return carry

    jax.lax.fori_loop(0, _BR, _wait, 0)
    g = sliver_ref[...]
    c_vec = jnp.minimum((t // _LANES) * _LANES, _CMAX)
    lane = t - c_vec
    lane_ids = jax.lax.broadcasted_iota(jnp.int32, (_BR, _LANES), 1)
    pt_sliver = jnp.sum(jnp.where(lane_ids == lane, g, 0.0), axis=1,
                        keepdims=True)
    pt = jnp.where(t >= _TAIL, pt_tail, pt_sliver)

    lse = m + jnp.log(es)
    logp_t = pt - lse
    logp_0 = p0 - lse
    s_row = rowsum - jnp.float32(_V) * lse
    contrib = _SMOOTH * (s_row - logp_t - logp_0) + _CONF * logp_t
    rmask = t != _IGNORE
    contrib = jnp.where(rmask, contrib, 0.0)
    csum = jnp.sum(contrib).reshape(1, 1)
    nv = jnp.sum(rmask.astype(jnp.float32)).reshape(1, 1)

    @pl.when(bi == 0)
    def _():
        acc_ref[...] = csum
        nv_ref[...] = nv

    @pl.when(bi > 0)
    def _():
        acc_ref[...] = acc_ref[...] + csum
        nv_ref[...] = nv_ref[...] + nv

    @pl.when(bi == nb - 1)
    def _():
        out_ref[...] = -acc_ref[...] / jnp.maximum(nv_ref[...], 1.0)


def kernel(pred, target):
    pred2 = pred.reshape(-1, pred.shape[-1])
    n = pred2.shape[0]
    t = target.reshape(n).astype(jnp.int32)
    nb = n // _BR

    grid_spec = pltpu.PrefetchScalarGridSpec(
        num_scalar_prefetch=1,
        grid=(nb,),
        in_specs=[
            pl.BlockSpec((_BR, _V), lambda b, t_sm: (b, 0)),
            pl.BlockSpec(memory_space=pltpu.MemorySpace.HBM),
            pl.BlockSpec((_BR, 1), lambda b, t_sm: (b, 0)),
        ],
        out_specs=pl.BlockSpec((1, 1), lambda b, t_sm: (0, 0)),
        scratch_shapes=[
            pltpu.VMEM((_BR, _LANES), jnp.float32),
            pltpu.SemaphoreType.DMA,
            pltpu.VMEM((1, 1), jnp.float32),
            pltpu.VMEM((1, 1), jnp.float32),
        ],
    )
    out = pl.pallas_call(
        _loss_kernel,
        grid_spec=grid_spec,
        out_shape=jax.ShapeDtypeStruct((1, 1), jnp.float32),
    )(t, pred2, pred2, t.reshape(n, 1))
    return out[0, 0]


# speedup vs baseline: 2.2194x; 1.0296x over previous
import jax
import jax.numpy as jnp
from jax.experimental import pallas as pl
from jax.experimental.pallas import tpu as pltpu

_V = 100000
_EPS = 0.1
_CONF = 1.0 - _EPS
_SMOOTH = _EPS / (_V - 2)
_IGNORE = 0

_LANES = 128
_VA = _V // _LANES * _LANES
_CMAX = (_V - 160) // _LANES * _LANES
_TAIL = _CMAX + _LANES
_BR = 32
_RPS = 2 * _BR


def _row_stats(x):
    xa = x[:, :_VA]
    xt = x[:, _VA:_V]
    m = jnp.maximum(jnp.max(xa, axis=1, keepdims=True),
                    jnp.max(xt, axis=1, keepdims=True))
    rowsum = (jnp.sum(xa, axis=1, keepdims=True)
              + jnp.sum(xt, axis=1, keepdims=True))
    es = (jnp.sum(jnp.exp(xa - m), axis=1, keepdims=True)
          + jnp.sum(jnp.exp(xt - m), axis=1, keepdims=True))
    return m, rowsum, es, x[:, 0:1], xt


def _loss_kernel(t_sm, x_ref, y_ref, pred_any, t_ref, out_ref,
                 sliver_ref, sem, acc_ref, nv_ref):
    bi = pl.program_id(0)
    nb = pl.num_programs(0)
    base = bi * _RPS

    def _sliver_copy(local):
        r = base + local
        c = jnp.minimum((t_sm[r] // _LANES) * _LANES, _CMAX)
        return pltpu.make_async_copy(
            pred_any.at[r, pl.ds(c, _LANES)], sliver_ref.at[local], sem)

    def _issue(local, carry):
        _sliver_copy(local).start()
        return carry

    jax.lax.fori_loop(0, _RPS, _issue, 0)

    mx, sx, ex, p0x, xt_x = _row_stats(x_ref[...])
    my, sy, ey, p0y, xt_y = _row_stats(y_ref[...])
    m = jnp.concatenate([mx, my], axis=0)
    rowsum = jnp.concatenate([sx, sy], axis=0)
    es = jnp.concatenate([ex, ey], axis=0)
    p0 = jnp.concatenate([p0x, p0y], axis=0)
    xt = jnp.concatenate([xt_x, xt_y], axis=0)

    t = t_ref[...]
    tail_ids = _VA + jax.lax.broadcasted_iota(jnp.int32, (_RPS, _V - _VA), 1)
    pt_tail = jnp.sum(jnp.where(tail_ids == t, xt, 0.0), axis=1,
                      keepdims=True)

    def _wait(local, carry):
        _sliver_copy(local).wait()
        return carry

    jax.lax.fori_loop(0, _RPS, _wait, 0)
    g = sliver_ref[...]
    c_vec = jnp.minimum((t // _LANES) * _LANES, _CMAX)
    lane = t - c_vec
    lane_ids = jax.lax.broadcasted_iota(jnp.int32, (_RPS, _LANES), 1)
    pt_sliver = jnp.sum(jnp.where(lane_ids == lane, g, 0.0), axis=1,
                        keepdims=True)
    pt = jnp.where(t >= _TAIL, pt_tail, pt_sliver)

    lse = m + jnp.log(es)
    logp_t = pt - lse
    logp_0 = p0 - lse
    s_row = rowsum - jnp.float32(_V) * lse
    contrib = _SMOOTH * (s_row - logp_t - logp_0) + _CONF * logp_t
    rmask = t != _IGNORE
    contrib = jnp.where(rmask, contrib, 0.0)
    csum = jnp.sum(contrib).reshape(1, 1)
    nv = jnp.sum(rmask.astype(jnp.float32)).reshape(1, 1)

    @pl.when(bi == 0)
    def _():
        acc_ref[...] = csum
        nv_ref[...] = nv

    @pl.when(bi > 0)
    def _():
        acc_ref[...] = acc_ref[...] + csum
        nv_ref[...] = nv_ref[...] + nv

    @pl.when(bi == nb - 1)
    def _():
        out_ref[...] = -acc_ref[...] / jnp.maximum(nv_ref[...], 1.0)


def kernel(pred, target):
    pred2 = pred.reshape(-1, pred.shape[-1])
    n = pred2.shape[0]
    t = target.reshape(n).astype(jnp.int32)
    nb = n // _RPS

    grid_spec = pltpu.PrefetchScalarGridSpec(
        num_scalar_prefetch=1,
        grid=(nb,),
        in_specs=[
            pl.BlockSpec((_BR, _V), lambda b, t_sm: (2 * b, 0)),
            pl.BlockSpec((_BR, _V), lambda b, t_sm: (2 * b + 1, 0)),
            pl.BlockSpec(memory_space=pltpu.MemorySpace.HBM),
            pl.BlockSpec((_RPS, 1), lambda b, t_sm: (b, 0)),
        ],
        out_specs=pl.BlockSpec((1, 1), lambda b, t_sm: (0, 0)),
        scratch_shapes=[
            pltpu.VMEM((_RPS, _LANES), jnp.float32),
            pltpu.SemaphoreType.DMA,
            pltpu.VMEM((1, 1), jnp.float32),
            pltpu.VMEM((1, 1), jnp.float32),
        ],
    )
    out = pl.pallas_call(
        _loss_kernel,
        grid_spec=grid_spec,
        out_shape=jax.ShapeDtypeStruct((1, 1), jnp.float32),
    )(t, pred2, pred2, pred2, t.reshape(n, 1))
    return out[0, 0]
